# baseline (speedup 1.0000x reference)
import functools

import jax
import jax.numpy as jnp
from jax import lax
from jax.experimental import pallas as pl
from jax.experimental.pallas import tpu as pltpu
from jax.experimental.pallas import tpu_sc as plsc

B = 16
F = 6
M = 8
K = 10
T = 32768
BT = 4096
NW = F * (K - 1)
NSUB = 16
TW = T // NSUB


def _tc_dense_body(xT_ref, rmat_ref, koff_ref, dmat_ref, bias_ref,
                   lbig_ref, pmat_ref, tok_ref):
    x = xT_ref[...]

    xr9 = jnp.dot(rmat_ref[...], x, preferred_element_type=jnp.float32)
    w = jnp.clip(xr9 - koff_ref[...], 0.0, 1.0)
    calib = jnp.dot(dmat_ref[...], w, preferred_element_type=jnp.float32)
    calib = jnp.clip(calib + bias_ref[...], 0.0, 1.0)
    X = [calib[f * M:(f + 1) * M] for f in range(F)]

    a1 = jnp.concatenate([1.0 - X[5], X[5]], 0)
    p2 = jnp.concatenate([a1[:M] * X[4], a1[M:] * X[4]], 0)
    a2 = jnp.concatenate([a1 - p2, p2], 0)
    p3 = jnp.concatenate([a2[i * M:(i + 1) * M] * X[3]
                          for i in range(4)], 0)
    a3 = jnp.concatenate([a2 - p3, p3], 0)

    V = jnp.dot(lbig_ref[...], a3, preferred_element_type=jnp.float32)

    d1 = V[32:] - V[:32]
    e1 = jnp.concatenate([d1[i * M:(i + 1) * M] * X[0]
                          for i in range(4)], 0)
    v32 = V[:32] + e1
    d2 = v32[16:] - v32[:16]
    e2 = jnp.concatenate([d2[:M] * X[1], d2[M:] * X[1]], 0)
    v16 = v32[:16] + e2
    d3 = v16[M:] - v16[:M]
    tok = v16[:M] + d3 * X[2]

    tok9 = jnp.concatenate([tok, jnp.ones((1, BT), jnp.float32)], 0)
    tok_ref[...] = jax.lax.dot_general(
        tok9, pmat_ref[...], (((0,), (0,)), ((), ())),
        preferred_element_type=jnp.float32)


def _run_tc_dense(xT, rmat, koff, dmat, bias, lbig, pmat):
    nblk = T // BT
    return pl.pallas_call(
        _tc_dense_body,
        grid=(nblk,),
        in_specs=[
            pl.BlockSpec((F, BT), lambda i: (0, i)),
            pl.BlockSpec((NW, F), lambda i: (0, 0)),
            pl.BlockSpec((NW, 1), lambda i: (0, 0)),
            pl.BlockSpec((F * M, NW), lambda i: (0, 0)),
            pl.BlockSpec((F * M, 1), lambda i: (0, 0)),
            pl.BlockSpec((64, 64), lambda i: (0, 0)),
            pl.BlockSpec((M + 1, 16), lambda i: (0, 0)),
        ],
        out_specs=pl.BlockSpec((BT, 16), lambda i: (i, 0)),
        out_shape=jax.ShapeDtypeStruct((T, 16), jnp.float32),
    )(xT, rmat, koff, dmat, bias, lbig, pmat)


def _sc_agg_body(tok_hbm, seg_hbm, acc_hbm, rows_v, seg_v, stage_v, acc_sh):
    c = lax.axis_index("c")
    s = lax.axis_index("s")

    @pl.when(c == 0)
    def _():
        @pl.when(s == 0)
        def _():
            for i in range(B):
                stage_v[i] = jnp.zeros((16,), jnp.float32)
            pltpu.sync_copy(stage_v, acc_sh)

        plsc.subcore_barrier()

        base = s * TW
        pltpu.sync_copy(tok_hbm.at[pl.ds(base, TW)], rows_v)
        pltpu.sync_copy(seg_hbm.at[pl.ds(base, TW)], seg_v)
        pltpu.sync_copy(rows_v, acc_sh.at[seg_v], add=True)

        plsc.subcore_barrier()

        @pl.when(s == 0)
        def _():
            pltpu.sync_copy(acc_sh, acc_hbm)


def _make_sc_agg():
    mesh = plsc.VectorSubcoreMesh(core_axis_name="c", subcore_axis_name="s")
    return pl.kernel(
        _sc_agg_body,
        mesh=mesh,
        compiler_params=pltpu.CompilerParams(use_tc_tiling_on_sc=False),
        out_type=jax.ShapeDtypeStruct((B, 16), jnp.float32),
        scratch_types=[
            pltpu.VMEM((TW, 16), jnp.float32),
            pltpu.VMEM((TW,), jnp.int32),
            pltpu.VMEM((B, 16), jnp.float32),
            pltpu.VMEM_SHARED((B, 16), jnp.float32),
        ],
    )


def _tc_tail_body(acc_ref, midkT_ref, fin_ref, out_ref):
    acc = acc_ref[...]
    midkT = midkT_ref[...]
    agg = acc[:, :M] / jnp.maximum(acc[:, M:M + 1], 1.0)
    mid = jnp.zeros((B, M), jnp.float32) + midkT[0:1, :]
    for k in range(K - 1):
        kp = -1.0 + 2.0 * k / (K - 1)
        wmk = jnp.clip((agg - kp) * ((K - 1) / 2.0), 0.0, 1.0)
        mid = mid + wmk * midkT[k + 1:k + 2, :]
    mid = jnp.clip(mid, 0.0, 1.0)
    vals = jnp.zeros((B, 2 ** M), jnp.float32) + fin_ref[...]
    half = (2 ** M) // 2
    for d in range(M):
        vals = (vals[:, :half]
                + (vals[:, half:] - vals[:, :half]) * mid[:, d:d + 1])
        half //= 2
    out_ref[...] = vals


def _run_tc_tail(acc, midkT, finr):
    return pl.pallas_call(
        _tc_tail_body,
        in_specs=[
            pl.BlockSpec((B, 16), lambda: (0, 0)),
            pl.BlockSpec((K, M), lambda: (0, 0)),
            pl.BlockSpec((1, 2 ** M), lambda: (0, 0)),
        ],
        out_specs=pl.BlockSpec((B, 1), lambda: (0, 0)),
        out_shape=jax.ShapeDtypeStruct((B, 1), jnp.float32),
    )(acc, midkT, finr)


@jax.jit
def _run(flat, segment_ids, calib_kernel, lattice_kernel, mid_kernel,
         final_kernel):
    xT = flat.T
    seg = segment_ids.astype(jnp.int32)

    frows = jnp.repeat(jnp.arange(F), K - 1)
    krows = jnp.tile(jnp.arange(K - 1), F)
    rmat = 9.0 * jax.nn.one_hot(frows, F, dtype=jnp.float32)
    koff = krows.astype(jnp.float32).reshape(NW, 1)
    deltas = calib_kernel[:, :, 1:]
    dmat = jnp.einsum('mfk,wf,wk->fmw',
                      deltas,
                      jax.nn.one_hot(frows, F, dtype=jnp.float32),
                      jax.nn.one_hot(krows, K - 1, dtype=jnp.float32)
                      ).reshape(F * M, NW)
    bias = calib_kernel[:, :, 0].T.reshape(F * M, 1)
    l3d = lattice_kernel.reshape(M, 8, 8)
    lbig = jnp.einsum('mpq,mn->pmqn', l3d,
                      jnp.eye(M, dtype=jnp.float32)).reshape(64, 64)
    pmat = jnp.eye(M + 1, 16, dtype=jnp.float32)

    tok_pad = _run_tc_dense(xT, rmat, koff, dmat, bias, lbig, pmat)
    acc = _make_sc_agg()(tok_pad, seg)
    return _run_tc_tail(acc, mid_kernel.T, final_kernel.reshape(1, 2 ** M))


def kernel(flat, segment_ids, calib_kernel, lattice_kernel, mid_kernel,
           final_kernel):
    return _run(flat, segment_ids, calib_kernel, lattice_kernel, mid_kernel,
                final_kernel)
